# baseline (speedup 1.0000x reference)
import functools

import jax
import jax.numpy as jnp
from jax.experimental import pallas as pl

F32 = jnp.float32
HI = jax.lax.Precision.HIGHEST


def _mm(a, b):
    return jax.lax.dot_general(a, b, (((1,), (0,)), ((), ())),
                               precision=HI, preferred_element_type=F32)


def _conv_body(x_ref, w_ref, b_ref, o_ref, *, k, dil, t_out, relu):
    acc = _mm(x_ref[0, 0:t_out, :], w_ref[0])
    for i in range(1, k):
        acc = acc + _mm(x_ref[0, i * dil:i * dil + t_out, :], w_ref[i])
    acc = acc + b_ref[...]
    if relu:
        acc = jnp.maximum(acc, 0.0)
    o_ref[0] = acc


def _conv(x, w, b, *, k, pad, dil=1, relu=False):
    bsz, t, cin = x.shape
    cout = w.shape[0]
    wt = jnp.transpose(w, (2, 1, 0))
    xp = jnp.pad(x, ((0, 0), (pad, pad), (0, 0)))
    tp = t + 2 * pad
    t_out = tp - dil * (k - 1)
    body = functools.partial(_conv_body, k=k, dil=dil, t_out=t_out, relu=relu)
    return pl.pallas_call(
        body,
        grid=(bsz,),
        in_specs=[pl.BlockSpec((1, tp, cin), lambda i: (i, 0, 0)),
                  pl.BlockSpec((k, cin, cout), lambda i: (0, 0, 0)),
                  pl.BlockSpec((1, cout), lambda i: (0, 0))],
        out_specs=pl.BlockSpec((1, t_out, cout), lambda i: (i, 0, 0)),
        out_shape=jax.ShapeDtypeStruct((bsz, t_out, cout), F32),
    )(xp, wt, b.reshape(1, cout))


def _res_body(x_ref, w1_ref, b1_ref, w2_ref, b2_ref, o_ref, *, d, t):
    hr = jnp.maximum(x_ref[0], 0.0)
    acc = _mm(hr[0:t, :], w1_ref[0])
    acc = acc + _mm(hr[d:d + t, :], w1_ref[1])
    acc = acc + _mm(hr[2 * d:2 * d + t, :], w1_ref[2])
    hh = jnp.maximum(acc + b1_ref[...], 0.0)
    o_ref[0] = x_ref[0, d:d + t, :] + (_mm(hh, w2_ref[0]) + b2_ref[...])


def _res(h, r, d):
    bsz, t, c = h.shape
    w1 = jnp.transpose(r['c1']['w'], (2, 1, 0))
    w2 = jnp.transpose(r['c2']['w'], (2, 1, 0))
    hp = jnp.pad(h, ((0, 0), (d, d), (0, 0)))
    body = functools.partial(_res_body, d=d, t=t)
    return pl.pallas_call(
        body,
        grid=(bsz,),
        in_specs=[pl.BlockSpec((1, t + 2 * d, c), lambda i: (i, 0, 0)),
                  pl.BlockSpec((3, c, c), lambda i: (0, 0, 0)),
                  pl.BlockSpec((1, c), lambda i: (0, 0)),
                  pl.BlockSpec((1, c, c), lambda i: (0, 0, 0)),
                  pl.BlockSpec((1, c), lambda i: (0, 0))],
        out_specs=pl.BlockSpec((1, t, c), lambda i: (i, 0, 0)),
        out_shape=jax.ShapeDtypeStruct((bsz, t, c), F32),
    )(hp, w1, r['c1']['b'].reshape(1, c), w2, r['c2']['b'].reshape(1, c))


def _down_body(e_ref, od_ref, w_ref, b_ref, o_ref, *, t2):
    acc = _mm(e_ref[0, 0:t2, :], w_ref[0])
    acc = acc + _mm(od_ref[0, 0:t2, :], w_ref[1])
    acc = acc + _mm(e_ref[0, 1:t2 + 1, :], w_ref[2])
    acc = acc + _mm(od_ref[0, 1:t2 + 1, :], w_ref[3])
    o_ref[0] = acc + b_ref[...]


def _down(x, w, b):
    bsz, t, c = x.shape
    t2 = t // 2
    wt = jnp.transpose(w, (2, 1, 0))
    xp = jnp.pad(x, ((0, 0), (1, 1), (0, 0)))
    ev = xp[:, 0::2, :]
    od = xp[:, 1::2, :]
    body = functools.partial(_down_body, t2=t2)
    return pl.pallas_call(
        body,
        grid=(bsz,),
        in_specs=[pl.BlockSpec((1, t2 + 1, c), lambda i: (i, 0, 0)),
                  pl.BlockSpec((1, t2 + 1, c), lambda i: (i, 0, 0)),
                  pl.BlockSpec((4, c, c), lambda i: (0, 0, 0)),
                  pl.BlockSpec((1, c), lambda i: (0, 0))],
        out_specs=pl.BlockSpec((1, t2, c), lambda i: (i, 0, 0)),
        out_shape=jax.ShapeDtypeStruct((bsz, t2, c), F32),
    )(ev, od, wt, b.reshape(1, c))


def _up_body(x_ref, w_ref, b_ref, ev_ref, od_ref, *, t):
    hp = x_ref[0]
    ev = _mm(hp[0:t, :], w_ref[0]) + _mm(hp[1:t + 1, :], w_ref[1] + w_ref[2])
    od = _mm(hp[1:t + 1, :], w_ref[0] + w_ref[1]) + _mm(hp[2:t + 2, :], w_ref[2])
    ev_ref[0] = ev + b_ref[...]
    od_ref[0] = od + b_ref[...]


def _up(x, w, b):
    bsz, t, c = x.shape
    wt = jnp.transpose(w, (2, 1, 0))
    xp = jnp.pad(x, ((0, 0), (1, 1), (0, 0)))
    body = functools.partial(_up_body, t=t)
    ev, od = pl.pallas_call(
        body,
        grid=(bsz,),
        in_specs=[pl.BlockSpec((1, t + 2, c), lambda i: (i, 0, 0)),
                  pl.BlockSpec((3, c, c), lambda i: (0, 0, 0)),
                  pl.BlockSpec((1, c), lambda i: (0, 0))],
        out_specs=[pl.BlockSpec((1, t, c), lambda i: (i, 0, 0)),
                   pl.BlockSpec((1, t, c), lambda i: (i, 0, 0))],
        out_shape=[jax.ShapeDtypeStruct((bsz, t, c), F32),
                   jax.ShapeDtypeStruct((bsz, t, c), F32)],
    )(xp, wt, b.reshape(1, c))
    return jnp.stack([ev, od], axis=2).reshape(bsz, 2 * t, c)


def _vq_body(z_ref, cb_ref, q_ref, i_ref, l_ref):
    z = z_ref[...]
    cb = cb_ref[...]
    zsq = jnp.sum(z * z, axis=1, keepdims=True)
    csq = jnp.sum(cb * cb, axis=1)[None, :]
    cross = jax.lax.dot_general(z, cb, (((1,), (1,)), ((), ())),
                                precision=HI, preferred_element_type=F32)
    d2 = zsq - 2.0 * cross + csq
    minv = jnp.min(d2, axis=1, keepdims=True)
    iota = jax.lax.broadcasted_iota(jnp.int32, d2.shape, 1)
    idx = jnp.min(jnp.where(d2 <= minv, iota, 2 ** 30), axis=1, keepdims=True)
    oh = (iota == idx).astype(F32)
    quant = _mm(oh, cb)
    dq = quant - z
    l_ref[...] = jnp.broadcast_to(5.0 * jnp.mean(dq * dq), (1, 1))
    q_ref[...] = z + dq
    i_ref[...] = idx


def _vq(z, codebook):
    bsz, t, d = z.shape
    n = bsz * t
    k = codebook.shape[0]
    flat = z.reshape(n, d)
    qst, idx, loss = pl.pallas_call(
        _vq_body,
        out_shape=[jax.ShapeDtypeStruct((n, d), F32),
                   jax.ShapeDtypeStruct((n, 1), jnp.int32),
                   jax.ShapeDtypeStruct((1, 1), F32)],
    )(flat, codebook)
    return qst.reshape(bsz, t, d), idx.reshape(bsz, t), loss.reshape(())


def kernel(motion, enc_params, codebook, dec_params):
    h = _conv(motion, enc_params['c0']['w'], enc_params['c0']['b'],
              k=3, pad=1, relu=True)
    for blk in enc_params['down']:
        h = _down(h, blk['cd']['w'], blk['cd']['b'])
        for j, r in enumerate(blk['res']):
            h = _res(h, r, 3 ** j)
    z = _conv(h, enc_params['cf']['w'], enc_params['cf']['b'], k=3, pad=1)

    qst, indices, commit_loss = _vq(z, codebook)

    h = _conv(qst, dec_params['c0']['w'], dec_params['c0']['b'],
              k=3, pad=1, relu=True)
    for blk in dec_params['up']:
        for j, r in enumerate(blk['res']):
            h = _res(h, r, 3 ** j)
        h = _up(h, blk['cu']['w'], blk['cu']['b'])
    h = _conv(h, dec_params['cf1']['w'], dec_params['cf1']['b'],
              k=3, pad=1, relu=True)
    decoded = _conv(h, dec_params['cf2']['w'], dec_params['cf2']['b'],
                    k=3, pad=1)
    return decoded, indices, commit_loss
